# baseline (speedup 1.0000x reference)
import jax
import jax.numpy as jnp
from jax.experimental import pallas as pl
from jax.experimental.pallas import tpu as pltpu

_B = 32
_D = 256
_H = 24
_W = 24
_HW = _H * _W
_BB = 4


def _pos_emb_kernel(p_ref, row_ref, col_ref, pose_ref, m_ref, pemb_ref, scratch_ref):
    pid = pl.program_id(0)

    @pl.when(pid == 0)
    def _():
        r = jax.lax.broadcasted_iota(jnp.int32, (_D, _HW), 0)
        l = jax.lax.broadcasted_iota(jnp.int32, (_D, _HW), 1)
        sel_col = (r == l % _W + 1).astype(jnp.float32)
        sel_row = (r == l // _W + 1).astype(jnp.float32)
        dn = (((0,), (0,)), ((), ()))
        top = jax.lax.dot_general(col_ref[...], sel_col, dn,
                                  preferred_element_type=jnp.float32)
        bot = jax.lax.dot_general(row_ref[...], sel_row, dn,
                                  preferred_element_type=jnp.float32)
        scratch_ref[0:_D, :] = top
        scratch_ref[_D:2 * _D, :] = bot

        onehot = (jax.lax.broadcasted_iota(jnp.int32, (8, _D), 1)
                  == p_ref[0]).astype(jnp.float32)
        pv = jax.lax.dot_general(onehot, pose_ref[...], (((1,), (0,)), ((), ())),
                                 preferred_element_type=jnp.float32)
        row = pv[0:1, :]
        pemb_ref[...] = jnp.broadcast_to(
            jnp.concatenate([row, row], axis=1), (_B, 2 * _D))

    m_ref[...] = jnp.broadcast_to(scratch_ref[...][None, :, :], (_BB, 2 * _D, _HW))


def kernel(x, row_W, col_W, pose_W, p):
    b, c, h, w = x.shape
    p_arr = jnp.asarray(p, dtype=jnp.int32).reshape((1,))
    m_flat, p_emb = pl.pallas_call(
        _pos_emb_kernel,
        grid=(_B // _BB,),
        in_specs=[
            pl.BlockSpec(memory_space=pltpu.SMEM),
            pl.BlockSpec((_D, _D), lambda i: (0, 0)),
            pl.BlockSpec((_D, _D), lambda i: (0, 0)),
            pl.BlockSpec((_D, _D), lambda i: (0, 0)),
        ],
        out_specs=[
            pl.BlockSpec((_BB, 2 * _D, _HW), lambda i: (i, 0, 0)),
            pl.BlockSpec((_B, 2 * _D), lambda i: (0, 0)),
        ],
        out_shape=[
            jax.ShapeDtypeStruct((_B, 2 * _D, _HW), jnp.float32),
            jax.ShapeDtypeStruct((_B, 2 * _D), jnp.float32),
        ],
        scratch_shapes=[pltpu.VMEM((2 * _D, _HW), jnp.float32)],
    )(p_arr, row_W, col_W, pose_W)
    return (p_emb, m_flat.reshape(b, 2 * _D, h, w))
